# baseline (speedup 1.0000x reference)
import functools

import jax
import jax.numpy as jnp
import numpy as np
from jax import lax
from jax.experimental import pallas as pl
from jax.experimental.pallas import tpu as pltpu
from jax.experimental.pallas import tpu_sc as plsc

_N = 10000
_E = 160000
_F_IN = 22
_F_E = 4
_H = 32
_EPS = 1e-5
_D = 128
_NPAD = 10240

_NC = 2
_NS = 16
_NW = _NC * _NS
_CHUNK = 128
_CPW = 40
_EPW = _CHUNK * _CPW
_EP = _EPW * _NW
_RPS = _NPAD // _NS

_BLK = 2048
_NBLK = _EP // _BLK


def _expand_mats(in_ch):
    ch = in_ch * _H
    r = np.zeros((_D, ch), np.float32)
    s = np.zeros((ch, _D), np.float32)
    for i in range(in_ch):
        r[i, i * _H:(i + 1) * _H] = 1.0
        s[i * _H:(i + 1) * _H, :_H] = np.eye(_H, dtype=np.float32)
    return r.astype(jnp.bfloat16), s.astype(jnp.bfloat16)


_R1, _S1 = _expand_mats(_F_IN)
_R2, _S2 = _expand_mats(_H)


@functools.cache
def _sc_mesh():
    return plsc.VectorSubcoreMesh(core_axis_name="c", subcore_axis_name="s",
                                  num_cores=_NC, num_subcores=_NS)


def _gather_body(table_hbm, idx_hbm, out_hbm, idx_v, rows0, rows1, sem0,
                 sem1):
    wid = lax.axis_index("s") * _NC + lax.axis_index("c")
    pltpu.sync_copy(idx_hbm.at[wid], idx_v)
    base = wid * _EPW
    bufs = ((rows0, sem0), (rows1, sem1))
    pltpu.async_copy(table_hbm.at[idx_v.at[0, 0]], rows0, sem0)
    pltpu.async_copy(table_hbm.at[idx_v.at[1, 0]], rows1, sem1)

    def pair(k, carry):
        for b in range(2):
            rows, sem = bufs[b]
            g = 2 * k + b
            pltpu.make_async_copy(table_hbm.at[idx_v.at[g, 0]], rows,
                                  sem).wait()
            pltpu.sync_copy(rows, out_hbm.at[pl.ds(base + g * _CHUNK, _CHUNK)])

            @pl.when(g + 2 < _CPW)
            def _(rows=rows, sem=sem, g=g):
                pltpu.async_copy(table_hbm.at[idx_v.at[g + 2, 0]], rows, sem)

        return carry

    lax.fori_loop(0, _CPW // 2, pair, 0)


@functools.cache
def _gather_kernel():
    return pl.kernel(
        _gather_body,
        out_type=jax.ShapeDtypeStruct((_EP, _D), jnp.float32),
        mesh=_sc_mesh(),
        scratch_types=[
            pltpu.VMEM((_CPW, 1, _CHUNK), jnp.int32),
            pltpu.VMEM((_CHUNK, _D), jnp.float32),
            pltpu.VMEM((_CHUNK, _D), jnp.float32),
            pltpu.SemaphoreType.DMA,
            pltpu.SemaphoreType.DMA,
        ],
    )


def _gather(table, idx):
    return _gather_kernel()(table, idx)


def _scatter_body(msg_hbm, idx_hbm, zeros_hbm, out_hbm, idx_v, rows0, rows1,
                  acc_sh, sem0, sem1):
    cid = lax.axis_index("c")
    sid = lax.axis_index("s")
    wid = sid * _NC + cid
    pltpu.sync_copy(zeros_hbm, acc_sh.at[pl.ds(sid * _RPS, _RPS)])
    plsc.subcore_barrier()
    pltpu.sync_copy(idx_hbm.at[wid], idx_v)
    base = wid * _EPW
    bufs = ((rows0, sem0), (rows1, sem1))
    pltpu.async_copy(msg_hbm.at[pl.ds(base, _CHUNK)], rows0, sem0)
    pltpu.async_copy(msg_hbm.at[pl.ds(base + _CHUNK, _CHUNK)], rows1, sem1)

    def pair(k, carry):
        for b in range(2):
            rows, sem = bufs[b]
            g = 2 * k + b
            pltpu.make_async_copy(
                msg_hbm.at[pl.ds(base + g * _CHUNK, _CHUNK)], rows,
                sem).wait()
            pltpu.sync_copy(rows, acc_sh.at[idx_v.at[g, 0]], add=True)

            @pl.when(g + 2 < _CPW)
            def _(rows=rows, sem=sem, g=g):
                pltpu.async_copy(
                    msg_hbm.at[pl.ds(base + (g + 2) * _CHUNK, _CHUNK)], rows,
                    sem)

        return carry

    lax.fori_loop(0, _CPW // 2, pair, 0)
    plsc.subcore_barrier()
    pltpu.sync_copy(acc_sh.at[pl.ds(sid * _RPS, _RPS)],
                    out_hbm.at[cid, pl.ds(sid * _RPS, _RPS)])


@functools.cache
def _scatter_kernel():
    return pl.kernel(
        _scatter_body,
        out_type=jax.ShapeDtypeStruct((_NC, _NPAD, _D), jnp.float32),
        mesh=_sc_mesh(),
        scratch_types=[
            pltpu.VMEM((_CPW, 1, _CHUNK), jnp.int32),
            pltpu.VMEM((_CHUNK, _D), jnp.float32),
            pltpu.VMEM((_CHUNK, _D), jnp.float32),
            pltpu.VMEM_SHARED((_NPAD, _D), jnp.float32),
            pltpu.SemaphoreType.DMA,
            pltpu.SemaphoreType.DMA,
        ],
    )


def _scatter(msg, idx, zeros):
    return _scatter_kernel()(msg, idx, zeros)


def _fused_body(eat_ref, xg_ref, w_ref, b_ref, r_ref, s_ref, out_ref):
    z = jnp.dot(eat_ref[...].T, w_ref[...],
                preferred_element_type=jnp.float32)
    z = jnp.maximum(z + b_ref[...], 0.0)
    xr = jnp.dot(xg_ref[...].astype(jnp.bfloat16), r_ref[...],
                 preferred_element_type=jnp.float32)
    p = (z * xr).astype(jnp.bfloat16)
    out_ref[...] = jnp.dot(p, s_ref[...],
                           preferred_element_type=jnp.float32)


def _fused_msgs(ea_t, xg, nn_w, nn_b, r, s):
    ch = nn_w.shape[1]
    return pl.pallas_call(
        _fused_body,
        grid=(_NBLK,),
        in_specs=[
            pl.BlockSpec((_F_E, _BLK), lambda i: (0, i)),
            pl.BlockSpec((_BLK, _D), lambda i: (i, 0)),
            pl.BlockSpec((_F_E, ch), lambda i: (0, 0)),
            pl.BlockSpec((1, ch), lambda i: (0, 0)),
            pl.BlockSpec((_D, ch), lambda i: (0, 0)),
            pl.BlockSpec((ch, _D), lambda i: (0, 0)),
        ],
        out_specs=pl.BlockSpec((_BLK, _D), lambda i: (i, 0)),
        out_shape=jax.ShapeDtypeStruct((_EP, _D), jnp.float32),
    )(ea_t, xg, nn_w, nn_b.reshape(1, ch), r, s)


def _bn1_body(aggp_ref, x_ref, w_ref, b_ref, g_ref, bt_ref, out_ref):
    agg = (aggp_ref[0] + aggp_ref[1])[:_N, :_H]
    pre = (agg + jnp.dot(x_ref[...], w_ref[...],
                         preferred_element_type=jnp.float32) + b_ref[...])
    m = jnp.mean(pre, axis=0, keepdims=True)
    v = jnp.mean((pre - m) * (pre - m), axis=0, keepdims=True)
    h = (pre - m) * lax.rsqrt(v + _EPS) * g_ref[...] + bt_ref[...]
    out_ref[:, :_H] = jnp.maximum(h, 0.0)
    out_ref[:, _H:] = jnp.zeros((_N, _D - _H), jnp.float32)


def _bn2_body(aggp_ref, h_ref, w_ref, b_ref, g_ref, bt_ref, fcw_ref, fcb_ref,
              out_ref):
    agg = (aggp_ref[0] + aggp_ref[1])[:_N, :_H]
    pre = (agg + jnp.dot(h_ref[...], w_ref[...],
                         preferred_element_type=jnp.float32) + b_ref[...])
    m = jnp.mean(pre, axis=0, keepdims=True)
    v = jnp.mean((pre - m) * (pre - m), axis=0, keepdims=True)
    h2 = (pre - m) * lax.rsqrt(v + _EPS) * g_ref[...] + bt_ref[...]
    h2 = jnp.maximum(h2, 0.0)
    logit = jnp.dot(h2, fcw_ref[...],
                    preferred_element_type=jnp.float32) + fcb_ref[...]
    out_ref[...] = 1.0 / (1.0 + jnp.exp(-logit))


def _bn1(aggp, x, w, b, g, bt):
    return pl.pallas_call(
        _bn1_body,
        out_shape=jax.ShapeDtypeStruct((_N, _D), jnp.float32),
    )(aggp, x, w, b.reshape(1, _H), g.reshape(1, _H), bt.reshape(1, _H))


def _bn2fc(aggp, h, w, b, g, bt, fcw, fcb):
    return pl.pallas_call(
        _bn2_body,
        out_shape=jax.ShapeDtypeStruct((_N, 1), jnp.float32),
    )(aggp, h, w, b.reshape(1, _H), g.reshape(1, _H), bt.reshape(1, _H),
      fcw, fcb.reshape(1, 1))


def kernel(x, edge_index, edge_attr, en1_W, en1_b, root1_W, bias1, en2_W,
           en2_b, root2_W, bias2, bn1_gamma, bn1_beta, bn2_gamma, bn2_beta,
           fc_W, fc_b):
    pad_e = _EP - _E
    src = jnp.pad(edge_index[0], (0, pad_e)).reshape(_NW, _CPW, 1, _CHUNK)
    dst = jnp.pad(edge_index[1], (0, pad_e),
                  constant_values=_N).reshape(_NW, _CPW, 1, _CHUNK)
    ea_t = jnp.pad(edge_attr.T, ((0, 0), (0, pad_e)))
    xpad = jnp.pad(x, ((0, 0), (0, _D - _F_IN)))
    root2_pad = jnp.pad(root2_W, ((0, _D - _H), (0, 0)))
    zeros = jnp.zeros((_RPS, _D), jnp.float32)

    xg = _gather(xpad, src)
    msg1 = _fused_msgs(ea_t, xg, en1_W, en1_b, _R1, _S1)
    aggp1 = _scatter(msg1, dst, zeros)
    h = _bn1(aggp1, x, root1_W, bias1, bn1_gamma, bn1_beta)

    hg = _gather(h, src)
    msg2 = _fused_msgs(ea_t, hg, en2_W, en2_b, _R2, _S2)
    aggp2 = _scatter(msg2, dst, zeros)
    return _bn2fc(aggp2, h, root2_pad, bias2, bn2_gamma, bn2_beta, fc_W, fc_b)


# speedup vs baseline: 1.2171x; 1.2171x over previous
import functools

import jax
import jax.numpy as jnp
import numpy as np
from jax import lax
from jax.experimental import pallas as pl
from jax.experimental.pallas import tpu as pltpu
from jax.experimental.pallas import tpu_sc as plsc

_N = 10000
_E = 160000
_F_IN = 22
_F_E = 4
_H = 32
_EPS = 1e-5
_D = 32
_NPAD = 10240

_NC = 2
_NS = 16
_NW = _NC * _NS
_CHUNK = 128
_CPW = 40
_EPW = _CHUNK * _CPW
_EP = _EPW * _NW
_EQ = _EP // 4
_QPC = _CHUNK // 4
_RPS = _NPAD // _NS

_BLKQ = 512
_NBLK = _EQ // _BLKQ


def _expand_mats(in_ch):
    ch = in_ch * _H
    r = np.zeros((128, 4 * ch), np.float32)
    s = np.zeros((4 * ch, 128), np.float32)
    for q in range(4):
        for i in range(in_ch):
            r[q * _D + i, q * ch + i * _H:q * ch + (i + 1) * _H] = 1.0
            s[q * ch + i * _H:q * ch + (i + 1) * _H,
              q * _D:q * _D + _H] = np.eye(_H, dtype=np.float32)
    return r.astype(jnp.bfloat16), s.astype(jnp.bfloat16)


_R1, _S1 = _expand_mats(_F_IN)
_R2, _S2 = _expand_mats(_H)


@functools.cache
def _sc_mesh():
    return plsc.VectorSubcoreMesh(core_axis_name="c", subcore_axis_name="s",
                                  num_cores=_NC, num_subcores=_NS)


def _gather_body(table_hbm, idx_hbm, out_hbm, idx_v, rows0, rows1, sem0,
                 sem1):
    wid = lax.axis_index("s") * _NC + lax.axis_index("c")
    out_e = out_hbm
    pltpu.sync_copy(idx_hbm.at[wid], idx_v)
    base = wid * _EPW
    bufs = ((rows0, sem0), (rows1, sem1))
    pltpu.async_copy(table_hbm.at[idx_v.at[0]], rows0, sem0)
    pltpu.async_copy(table_hbm.at[idx_v.at[1]], rows1, sem1)

    def pair(k, carry):
        for b in range(2):
            rows, sem = bufs[b]
            g = 2 * k + b
            pltpu.make_async_copy(table_hbm.at[idx_v.at[g]], rows,
                                  sem).wait()
            pltpu.sync_copy(rows, out_e.at[pl.ds(base + g * _CHUNK, _CHUNK)])

            @pl.when(g + 2 < _CPW)
            def _(rows=rows, sem=sem, g=g):
                pltpu.async_copy(table_hbm.at[idx_v.at[g + 2]], rows, sem)

        return carry

    lax.fori_loop(0, _CPW // 2, pair, 0)


@functools.cache
def _gather_kernel():
    return pl.kernel(
        _gather_body,
        out_type=jax.ShapeDtypeStruct((_EP, _D), jnp.float32),
        mesh=_sc_mesh(),
        compiler_params=pltpu.CompilerParams(use_tc_tiling_on_sc=False),
        scratch_types=[
            pltpu.VMEM((_CPW, _CHUNK), jnp.int32),
            pltpu.VMEM((_CHUNK, _D), jnp.float32),
            pltpu.VMEM((_CHUNK, _D), jnp.float32),
            pltpu.SemaphoreType.DMA,
            pltpu.SemaphoreType.DMA,
        ],
    )


def _gather(table, idx):
    return _gather_kernel()(table, idx)


def _scatter_body(msg_hbm, idx_hbm, zeros_hbm, out_hbm, idx_v, rows0, rows1,
                  acc_sh, sem0, sem1):
    cid = lax.axis_index("c")
    sid = lax.axis_index("s")
    wid = sid * _NC + cid
    pltpu.sync_copy(zeros_hbm, acc_sh.at[pl.ds(sid * _RPS, _RPS)])
    plsc.subcore_barrier()
    msg_e = msg_hbm
    pltpu.sync_copy(idx_hbm.at[wid], idx_v)
    base = wid * _EPW
    bufs = ((rows0, sem0), (rows1, sem1))
    pltpu.async_copy(msg_e.at[pl.ds(base, _CHUNK)], rows0, sem0)
    pltpu.async_copy(msg_e.at[pl.ds(base + _CHUNK, _CHUNK)], rows1, sem1)

    def pair(k, carry):
        for b in range(2):
            rows, sem = bufs[b]
            g = 2 * k + b
            pltpu.make_async_copy(
                msg_e.at[pl.ds(base + g * _CHUNK, _CHUNK)], rows, sem).wait()
            pltpu.sync_copy(rows, acc_sh.at[idx_v.at[g]], add=True)

            @pl.when(g + 2 < _CPW)
            def _(rows=rows, sem=sem, g=g):
                pltpu.async_copy(
                    msg_e.at[pl.ds(base + (g + 2) * _CHUNK, _CHUNK)], rows,
                    sem)

        return carry

    lax.fori_loop(0, _CPW // 2, pair, 0)
    plsc.subcore_barrier()
    pltpu.sync_copy(acc_sh.at[pl.ds(sid * _RPS, _RPS)],
                    out_hbm.at[cid, pl.ds(sid * _RPS, _RPS)])


@functools.cache
def _scatter_kernel():
    return pl.kernel(
        _scatter_body,
        out_type=jax.ShapeDtypeStruct((_NC, _NPAD, _D), jnp.float32),
        mesh=_sc_mesh(),
        compiler_params=pltpu.CompilerParams(use_tc_tiling_on_sc=False),
        scratch_types=[
            pltpu.VMEM((_CPW, _CHUNK), jnp.int32),
            pltpu.VMEM((_CHUNK, _D), jnp.float32),
            pltpu.VMEM((_CHUNK, _D), jnp.float32),
            pltpu.VMEM_SHARED((_NPAD, _D), jnp.float32),
            pltpu.SemaphoreType.DMA,
            pltpu.SemaphoreType.DMA,
        ],
    )


def _scatter(msg, idx, zeros):
    return _scatter_kernel()(msg, idx, zeros)


def _fused_body(ea4_ref, xg4_ref, w4_ref, b4_ref, r4_ref, s4_ref, out_ref):
    z = jnp.dot(ea4_ref[...], w4_ref[...], preferred_element_type=jnp.float32)
    z = jnp.maximum(z + b4_ref[...], 0.0)
    xr = jnp.dot(xg4_ref[...].astype(jnp.bfloat16), r4_ref[...],
                 preferred_element_type=jnp.float32)
    p = (z * xr).astype(jnp.bfloat16)
    out_ref[...] = jnp.dot(p, s4_ref[...],
                           preferred_element_type=jnp.float32)


def _fused_msgs(ea4, xg4, w4, b4, r4, s4):
    ch4 = w4.shape[1]
    return pl.pallas_call(
        _fused_body,
        grid=(_NBLK,),
        in_specs=[
            pl.BlockSpec((_BLKQ, 4 * _F_E), lambda i: (i, 0)),
            pl.BlockSpec((_BLKQ, 128), lambda i: (i, 0)),
            pl.BlockSpec((4 * _F_E, ch4), lambda i: (0, 0)),
            pl.BlockSpec((1, ch4), lambda i: (0, 0)),
            pl.BlockSpec((128, ch4), lambda i: (0, 0)),
            pl.BlockSpec((ch4, 128), lambda i: (0, 0)),
        ],
        out_specs=pl.BlockSpec((_BLKQ, 128), lambda i: (i, 0)),
        out_shape=jax.ShapeDtypeStruct((_EQ, 128), jnp.float32),
    )(ea4, xg4, w4, b4, r4, s4)


def _bn1_body(aggp_ref, x_ref, w_ref, b_ref, g_ref, bt_ref, out_ref):
    agg = (aggp_ref[0] + aggp_ref[1])[:_N]
    pre = (agg + jnp.dot(x_ref[...], w_ref[...],
                         preferred_element_type=jnp.float32) + b_ref[...])
    m = jnp.mean(pre, axis=0, keepdims=True)
    v = jnp.mean((pre - m) * (pre - m), axis=0, keepdims=True)
    h = (pre - m) * lax.rsqrt(v + _EPS) * g_ref[...] + bt_ref[...]
    out_ref[...] = jnp.maximum(h, 0.0)


def _bn2_body(aggp_ref, h_ref, w_ref, b_ref, g_ref, bt_ref, fcw_ref, fcb_ref,
              out_ref):
    agg = (aggp_ref[0] + aggp_ref[1])[:_N]
    pre = (agg + jnp.dot(h_ref[...], w_ref[...],
                         preferred_element_type=jnp.float32) + b_ref[...])
    m = jnp.mean(pre, axis=0, keepdims=True)
    v = jnp.mean((pre - m) * (pre - m), axis=0, keepdims=True)
    h2 = (pre - m) * lax.rsqrt(v + _EPS) * g_ref[...] + bt_ref[...]
    h2 = jnp.maximum(h2, 0.0)
    logit = jnp.dot(h2, fcw_ref[...],
                    preferred_element_type=jnp.float32) + fcb_ref[...]
    out_ref[...] = 1.0 / (1.0 + jnp.exp(-logit))


def _bn1(aggp, x, w, b, g, bt):
    return pl.pallas_call(
        _bn1_body,
        out_shape=jax.ShapeDtypeStruct((_N, _H), jnp.float32),
    )(aggp, x, w, b.reshape(1, _H), g.reshape(1, _H), bt.reshape(1, _H))


def _bn2fc(aggp, h, w, b, g, bt, fcw, fcb):
    return pl.pallas_call(
        _bn2_body,
        out_shape=jax.ShapeDtypeStruct((_N, 1), jnp.float32),
    )(aggp, h, w, b.reshape(1, _H), g.reshape(1, _H), bt.reshape(1, _H),
      fcw, fcb.reshape(1, 1))


def kernel(x, edge_index, edge_attr, en1_W, en1_b, root1_W, bias1, en2_W,
           en2_b, root2_W, bias2, bn1_gamma, bn1_beta, bn2_gamma, bn2_beta,
           fc_W, fc_b):
    pad_e = _EP - _E
    src = jnp.pad(edge_index[0], (0, pad_e)).reshape(_NW, _CPW, _CHUNK)
    dst = jnp.pad(edge_index[1], (0, pad_e),
                  constant_values=_N).reshape(_NW, _CPW, _CHUNK)
    ea4 = jnp.pad(edge_attr, ((0, pad_e), (0, 0))).reshape(_EQ, 4 * _F_E)
    xpad = jnp.pad(x, ((0, 0), (0, _D - _F_IN)))
    zeros = jnp.zeros((_RPS, _D), jnp.float32)
    eye4 = jnp.eye(4, dtype=jnp.float32)
    w4_1 = jnp.kron(eye4, en1_W)
    b4_1 = jnp.tile(en1_b, 4).reshape(1, -1)
    w4_2 = jnp.kron(eye4, en2_W)
    b4_2 = jnp.tile(en2_b, 4).reshape(1, -1)

    xg = _gather(xpad, src)
    msg1 = _fused_msgs(ea4, xg.reshape(_EQ, 128), w4_1, b4_1, _R1, _S1)
    aggp1 = _scatter(msg1.reshape(_EP, _D), dst, zeros)
    h = _bn1(aggp1, x, root1_W, bias1, bn1_gamma, bn1_beta)

    hg = _gather(h, src)
    msg2 = _fused_msgs(ea4, hg.reshape(_EQ, 128), w4_2, b4_2, _R2, _S2)
    aggp2 = _scatter(msg2.reshape(_EP, _D), dst, zeros)
    return _bn2fc(aggp2, h, root2_W, bias2, bn2_gamma, bn2_beta, fc_W, fc_b)


# speedup vs baseline: 1.2670x; 1.0410x over previous
import functools

import jax
import jax.numpy as jnp
import numpy as np
from jax import lax
from jax.experimental import pallas as pl
from jax.experimental.pallas import tpu as pltpu
from jax.experimental.pallas import tpu_sc as plsc

_N = 10000
_E = 160000
_F_IN = 22
_F_E = 4
_H = 32
_EPS = 1e-5
_D = 32
_NPAD = 10240

_NC = 2
_NS = 16
_NW = _NC * _NS
_CHUNK = 128
_CPW = 40
_EPW = _CHUNK * _CPW
_EP = _EPW * _NW
_EQ = _EP // 4
_QPC = _CHUNK // 4
_RPS = _NPAD // _NS

_BLKQ = 512
_NBLK = _EQ // _BLKQ


def _expand_mats(in_ch):
    ch = in_ch * _H
    r = np.zeros((128, 4 * ch), np.float32)
    s = np.zeros((4 * ch, 128), np.float32)
    for q in range(4):
        for i in range(in_ch):
            r[q * _D + i, q * ch + i * _H:q * ch + (i + 1) * _H] = 1.0
            s[q * ch + i * _H:q * ch + (i + 1) * _H,
              q * _D:q * _D + _H] = np.eye(_H, dtype=np.float32)
    return r.astype(jnp.bfloat16), s.astype(jnp.bfloat16)


_R1, _S1 = _expand_mats(_F_IN)
_R2, _S2 = _expand_mats(_H)


@functools.cache
def _sc_mesh():
    return plsc.VectorSubcoreMesh(core_axis_name="c", subcore_axis_name="s",
                                  num_cores=_NC, num_subcores=_NS)


def _gather_body(table_hbm, idx_hbm, out_hbm, idx_v, rows0, rows1, sem0,
                 sem1):
    wid = lax.axis_index("s") * _NC + lax.axis_index("c")
    out_e = out_hbm
    pltpu.sync_copy(idx_hbm.at[wid], idx_v)
    base = wid * _EPW
    bufs = ((rows0, sem0), (rows1, sem1))
    pltpu.async_copy(table_hbm.at[idx_v.at[0]], rows0, sem0)
    pltpu.async_copy(table_hbm.at[idx_v.at[1]], rows1, sem1)

    def pair(k, carry):
        for b in range(2):
            rows, sem = bufs[b]
            g = 2 * k + b
            pltpu.make_async_copy(table_hbm.at[idx_v.at[g]], rows,
                                  sem).wait()
            pltpu.sync_copy(rows, out_e.at[pl.ds(base + g * _CHUNK, _CHUNK)])

            @pl.when(g + 2 < _CPW)
            def _(rows=rows, sem=sem, g=g):
                pltpu.async_copy(table_hbm.at[idx_v.at[g + 2]], rows, sem)

        return carry

    lax.fori_loop(0, _CPW // 2, pair, 0)


@functools.cache
def _gather_kernel():
    return pl.kernel(
        _gather_body,
        out_type=jax.ShapeDtypeStruct((_EP, _D), jnp.float32),
        mesh=_sc_mesh(),
        compiler_params=pltpu.CompilerParams(use_tc_tiling_on_sc=False),
        scratch_types=[
            pltpu.VMEM((_CPW, _CHUNK), jnp.int32),
            pltpu.VMEM((_CHUNK, _D), jnp.float32),
            pltpu.VMEM((_CHUNK, _D), jnp.float32),
            pltpu.SemaphoreType.DMA,
            pltpu.SemaphoreType.DMA,
        ],
    )


def _gather(table, idx):
    return _gather_kernel()(table, idx)


def _scatter_body(msg_hbm, idx_hbm, zeros_hbm, out_hbm, idx_v, rows0, rows1,
                  acc_sh, sem0, sem1):
    cid = lax.axis_index("c")
    sid = lax.axis_index("s")
    wid = sid * _NC + cid
    pltpu.sync_copy(zeros_hbm, acc_sh.at[pl.ds(sid * _RPS, _RPS)])
    plsc.subcore_barrier()
    msg_e = msg_hbm
    pltpu.sync_copy(idx_hbm.at[wid], idx_v)
    base = wid * _EPW
    bufs = ((rows0, sem0), (rows1, sem1))
    pltpu.async_copy(msg_e.at[pl.ds(base, _CHUNK)], rows0, sem0)
    pltpu.async_copy(msg_e.at[pl.ds(base + _CHUNK, _CHUNK)], rows1, sem1)

    def pair(k, carry):
        for b in range(2):
            rows, sem = bufs[b]
            g = 2 * k + b
            pltpu.make_async_copy(
                msg_e.at[pl.ds(base + g * _CHUNK, _CHUNK)], rows, sem).wait()
            pltpu.sync_copy(rows, acc_sh.at[idx_v.at[g]], add=True)

            @pl.when(g + 2 < _CPW)
            def _(rows=rows, sem=sem, g=g):
                pltpu.async_copy(
                    msg_e.at[pl.ds(base + (g + 2) * _CHUNK, _CHUNK)], rows,
                    sem)

        return carry

    lax.fori_loop(0, _CPW // 2, pair, 0)
    plsc.subcore_barrier()
    pltpu.sync_copy(acc_sh.at[pl.ds(sid * _RPS, _RPS)],
                    out_hbm.at[cid, pl.ds(sid * _RPS, _RPS)])


@functools.cache
def _scatter_kernel():
    return pl.kernel(
        _scatter_body,
        out_type=jax.ShapeDtypeStruct((_NC, _NPAD, _D), jnp.float32),
        mesh=_sc_mesh(),
        compiler_params=pltpu.CompilerParams(use_tc_tiling_on_sc=False),
        scratch_types=[
            pltpu.VMEM((_CPW, _CHUNK), jnp.int32),
            pltpu.VMEM((_CHUNK, _D), jnp.float32),
            pltpu.VMEM((_CHUNK, _D), jnp.float32),
            pltpu.VMEM_SHARED((_NPAD, _D), jnp.float32),
            pltpu.SemaphoreType.DMA,
            pltpu.SemaphoreType.DMA,
        ],
    )


def _scatter(msg, idx, zeros):
    return _scatter_kernel()(msg, idx, zeros)


def _fused_body(ea4_ref, xg4_ref, w4_ref, b4_ref, r4_ref, s4_ref, out_ref):
    z = jnp.dot(ea4_ref[...], w4_ref[...], preferred_element_type=jnp.float32)
    z = jnp.maximum(z + b4_ref[...], 0.0)
    xr = jnp.dot(xg4_ref[...].astype(jnp.bfloat16), r4_ref[...],
                 preferred_element_type=jnp.float32)
    p = (z * xr).astype(jnp.bfloat16)
    out_ref[...] = jnp.dot(p, s4_ref[...],
                           preferred_element_type=jnp.float32)


def _fused_msgs(ea4, xg4, w4, b4, r4, s4):
    ch4 = w4.shape[1]
    return pl.pallas_call(
        _fused_body,
        grid=(_NBLK,),
        in_specs=[
            pl.BlockSpec((_BLKQ, 4 * _F_E), lambda i: (i, 0)),
            pl.BlockSpec((_BLKQ, 128), lambda i: (i, 0)),
            pl.BlockSpec((4 * _F_E, ch4), lambda i: (0, 0)),
            pl.BlockSpec((1, ch4), lambda i: (0, 0)),
            pl.BlockSpec((128, ch4), lambda i: (0, 0)),
            pl.BlockSpec((ch4, 128), lambda i: (0, 0)),
        ],
        out_specs=pl.BlockSpec((_BLKQ, 128), lambda i: (i, 0)),
        out_shape=jax.ShapeDtypeStruct((_EQ, 128), jnp.float32),
    )(ea4, xg4, w4, b4, r4, s4)


def _bn1_body(aggp_ref, x_ref, w_ref, b_ref, g_ref, bt_ref, out_ref):
    agg = (aggp_ref[0] + aggp_ref[1])[:_N]
    pre = (agg + jnp.dot(x_ref[...], w_ref[...],
                         preferred_element_type=jnp.float32) + b_ref[...])
    m = jnp.mean(pre, axis=0, keepdims=True)
    v = jnp.mean((pre - m) * (pre - m), axis=0, keepdims=True)
    h = (pre - m) * lax.rsqrt(v + _EPS) * g_ref[...] + bt_ref[...]
    out_ref[...] = jnp.maximum(h, 0.0)


def _bn2_body(aggp_ref, h_ref, w_ref, b_ref, g_ref, bt_ref, fcw_ref, fcb_ref,
              out_ref):
    agg = (aggp_ref[0] + aggp_ref[1])[:_N]
    pre = (agg + jnp.dot(h_ref[...], w_ref[...],
                         preferred_element_type=jnp.float32) + b_ref[...])
    m = jnp.mean(pre, axis=0, keepdims=True)
    v = jnp.mean((pre - m) * (pre - m), axis=0, keepdims=True)
    h2 = (pre - m) * lax.rsqrt(v + _EPS) * g_ref[...] + bt_ref[...]
    h2 = jnp.maximum(h2, 0.0)
    logit = jnp.dot(h2, fcw_ref[...],
                    preferred_element_type=jnp.float32) + fcb_ref[...]
    out_ref[...] = 1.0 / (1.0 + jnp.exp(-logit))


def _bn1(aggp, x, w, b, g, bt):
    return pl.pallas_call(
        _bn1_body,
        out_shape=jax.ShapeDtypeStruct((_N, _H), jnp.float32),
    )(aggp, x, w, b.reshape(1, _H), g.reshape(1, _H), bt.reshape(1, _H))


def _bn2fc(aggp, h, w, b, g, bt, fcw, fcb):
    return pl.pallas_call(
        _bn2_body,
        out_shape=jax.ShapeDtypeStruct((_N, 1), jnp.float32),
    )(aggp, h, w, b.reshape(1, _H), g.reshape(1, _H), bt.reshape(1, _H),
      fcw, fcb.reshape(1, 1))


def kernel(x, edge_index, edge_attr, en1_W, en1_b, root1_W, bias1, en2_W,
           en2_b, root2_W, bias2, bn1_gamma, bn1_beta, bn2_gamma, bn2_beta,
           fc_W, fc_b):
    pad_e = _EP - _E
    src = jnp.pad(edge_index[0], (0, pad_e)).reshape(_NW, _CPW, _CHUNK)
    dst = jnp.pad(edge_index[1], (0, pad_e),
                  constant_values=_N).reshape(_NW, _CPW, _CHUNK)
    ea4 = jnp.pad(edge_attr, ((0, pad_e), (0, 0))).reshape(
        _EQ, 4 * _F_E).astype(jnp.bfloat16)
    xpad = jnp.pad(x, ((0, 0), (0, _D - _F_IN)))
    zeros = jnp.zeros((_RPS, _D), jnp.float32)
    eye4 = jnp.eye(4, dtype=jnp.float32)
    w4_1 = jnp.kron(eye4, en1_W).astype(jnp.bfloat16)
    b4_1 = jnp.tile(en1_b, 4).reshape(1, -1)
    w4_2 = jnp.kron(eye4, en2_W).astype(jnp.bfloat16)
    b4_2 = jnp.tile(en2_b, 4).reshape(1, -1)

    xg = _gather(xpad, src)
    msg1 = _fused_msgs(ea4, xg.reshape(_EQ, 128), w4_1, b4_1, _R1, _S1)
    aggp1 = _scatter(msg1.reshape(_EP, _D), dst, zeros)
    h = _bn1(aggp1, x, root1_W, bias1, bn1_gamma, bn1_beta)

    hg = _gather(h, src)
    msg2 = _fused_msgs(ea4, hg.reshape(_EQ, 128), w4_2, b4_2, _R2, _S2)
    aggp2 = _scatter(msg2.reshape(_EP, _D), dst, zeros)
    return _bn2fc(aggp2, h, root2_W, bias2, bn2_gamma, bn2_beta, fc_W, fc_b)


# speedup vs baseline: 1.3325x; 1.0517x over previous
import functools

import jax
import jax.numpy as jnp
import numpy as np
from jax import lax
from jax.experimental import pallas as pl
from jax.experimental.pallas import tpu as pltpu
from jax.experimental.pallas import tpu_sc as plsc

_N = 10000
_E = 160000
_F_IN = 22
_F_E = 4
_H = 32
_EPS = 1e-5
_D = 32
_NPAD = 10240

_NC = 2
_NS = 16
_NW = _NC * _NS
_CHUNK = 128
_CPW = 40
_EPW = _CHUNK * _CPW
_EP = _EPW * _NW
_EQ = _EP // 4
_QPC = _CHUNK // 4
_RPS = _NPAD // _NS

_EH = _EP // 2
_CPW2 = _CPW // 2
_EQ2 = _EH // 4

_BLKQ = 512
_NBLK = _EQ2 // _BLKQ


def _expand_mats(in_ch):
    ch = in_ch * _H
    r = np.zeros((128, 4 * ch), np.float32)
    s = np.zeros((4 * ch, 128), np.float32)
    for q in range(4):
        for i in range(in_ch):
            r[q * _D + i, q * ch + i * _H:q * ch + (i + 1) * _H] = 1.0
            s[q * ch + i * _H:q * ch + (i + 1) * _H,
              q * _D:q * _D + _H] = np.eye(_H, dtype=np.float32)
    return r.astype(jnp.bfloat16), s.astype(jnp.bfloat16)


_R1, _S1 = _expand_mats(_F_IN)
_R2, _S2 = _expand_mats(_H)


@functools.cache
def _sc_mesh():
    return plsc.VectorSubcoreMesh(core_axis_name="c", subcore_axis_name="s",
                                  num_cores=_NC, num_subcores=_NS)


@functools.cache
def _gather_kernel(cpw):
    def body(table_hbm, idx_hbm, out_hbm, idx_v, rows0, rows1, sem0, sem1):
        wid = lax.axis_index("s") * _NC + lax.axis_index("c")
        pltpu.sync_copy(idx_hbm.at[wid], idx_v)
        base = wid * cpw * _CHUNK
        bufs = ((rows0, sem0), (rows1, sem1))
        pltpu.async_copy(table_hbm.at[idx_v.at[0]], rows0, sem0)
        pltpu.async_copy(table_hbm.at[idx_v.at[1]], rows1, sem1)

        def pair(k, carry):
            for b in range(2):
                rows, sem = bufs[b]
                g = 2 * k + b
                pltpu.make_async_copy(table_hbm.at[idx_v.at[g]], rows,
                                      sem).wait()
                pltpu.sync_copy(rows,
                                out_hbm.at[pl.ds(base + g * _CHUNK, _CHUNK)])

                @pl.when(g + 2 < cpw)
                def _(rows=rows, sem=sem, g=g):
                    pltpu.async_copy(table_hbm.at[idx_v.at[g + 2]], rows, sem)

            return carry

        lax.fori_loop(0, cpw // 2, pair, 0)

    return pl.kernel(
        body,
        out_type=jax.ShapeDtypeStruct((_NW * cpw * _CHUNK, _D), jnp.float32),
        mesh=_sc_mesh(),
        compiler_params=pltpu.CompilerParams(use_tc_tiling_on_sc=False),
        scratch_types=[
            pltpu.VMEM((cpw, _CHUNK), jnp.int32),
            pltpu.VMEM((_CHUNK, _D), jnp.float32),
            pltpu.VMEM((_CHUNK, _D), jnp.float32),
            pltpu.SemaphoreType.DMA,
            pltpu.SemaphoreType.DMA,
        ],
    )


def _gather(table, idx):
    return _gather_kernel(idx.shape[1])(table, idx)


@functools.cache
def _scatter_kernel(cpw):
    def body(msg_hbm, idx_hbm, zeros_hbm, out_hbm, idx_v, rows0, rows1,
             acc_sh, sem0, sem1):
        cid = lax.axis_index("c")
        sid = lax.axis_index("s")
        wid = sid * _NC + cid
        pltpu.sync_copy(zeros_hbm, acc_sh.at[pl.ds(sid * _RPS, _RPS)])
        plsc.subcore_barrier()
        pltpu.sync_copy(idx_hbm.at[wid], idx_v)
        base = wid * cpw * _CHUNK
        bufs = ((rows0, sem0), (rows1, sem1))
        pltpu.async_copy(msg_hbm.at[pl.ds(base, _CHUNK)], rows0, sem0)
        pltpu.async_copy(msg_hbm.at[pl.ds(base + _CHUNK, _CHUNK)], rows1,
                         sem1)

        def pair(k, carry):
            for b in range(2):
                rows, sem = bufs[b]
                g = 2 * k + b
                pltpu.make_async_copy(
                    msg_hbm.at[pl.ds(base + g * _CHUNK, _CHUNK)], rows,
                    sem).wait()
                pltpu.sync_copy(rows, acc_sh.at[idx_v.at[g]], add=True)

                @pl.when(g + 2 < cpw)
                def _(rows=rows, sem=sem, g=g):
                    pltpu.async_copy(
                        msg_hbm.at[pl.ds(base + (g + 2) * _CHUNK, _CHUNK)],
                        rows, sem)

            return carry

        lax.fori_loop(0, cpw // 2, pair, 0)
        plsc.subcore_barrier()
        pltpu.sync_copy(acc_sh.at[pl.ds(sid * _RPS, _RPS)],
                        out_hbm.at[cid, pl.ds(sid * _RPS, _RPS)])

    return pl.kernel(
        body,
        out_type=jax.ShapeDtypeStruct((_NC, _NPAD, _D), jnp.float32),
        mesh=_sc_mesh(),
        compiler_params=pltpu.CompilerParams(use_tc_tiling_on_sc=False),
        scratch_types=[
            pltpu.VMEM((cpw, _CHUNK), jnp.int32),
            pltpu.VMEM((_CHUNK, _D), jnp.float32),
            pltpu.VMEM((_CHUNK, _D), jnp.float32),
            pltpu.VMEM_SHARED((_NPAD, _D), jnp.float32),
            pltpu.SemaphoreType.DMA,
            pltpu.SemaphoreType.DMA,
        ],
    )


def _scatter(msg, idx, zeros):
    return _scatter_kernel(idx.shape[1])(msg, idx, zeros)


def _fused_body(ea4_ref, xg4_ref, w4_ref, b4_ref, r4_ref, s4_ref, out_ref):
    z = jnp.dot(ea4_ref[...], w4_ref[...], preferred_element_type=jnp.float32)
    z = jnp.maximum(z + b4_ref[...], 0.0)
    xr = jnp.dot(xg4_ref[...].astype(jnp.bfloat16), r4_ref[...],
                 preferred_element_type=jnp.float32)
    p = (z * xr).astype(jnp.bfloat16)
    out_ref[...] = jnp.dot(p, s4_ref[...],
                           preferred_element_type=jnp.float32)


def _fused_msgs(ea4, xg4, w4, b4, r4, s4):
    ch4 = w4.shape[1]
    return pl.pallas_call(
        _fused_body,
        grid=(_NBLK,),
        in_specs=[
            pl.BlockSpec((_BLKQ, 4 * _F_E), lambda i: (i, 0)),
            pl.BlockSpec((_BLKQ, 128), lambda i: (i, 0)),
            pl.BlockSpec((4 * _F_E, ch4), lambda i: (0, 0)),
            pl.BlockSpec((1, ch4), lambda i: (0, 0)),
            pl.BlockSpec((128, ch4), lambda i: (0, 0)),
            pl.BlockSpec((ch4, 128), lambda i: (0, 0)),
        ],
        out_specs=pl.BlockSpec((_BLKQ, 128), lambda i: (i, 0)),
        out_shape=jax.ShapeDtypeStruct((_EQ2, 128), jnp.float32),
    )(ea4, xg4, w4, b4, r4, s4)


def _bn1_body(aggp_ref, aggq_ref, x_ref, w_ref, b_ref, g_ref, bt_ref,
              out_ref):
    agg = (aggp_ref[0] + aggp_ref[1] + aggq_ref[0] + aggq_ref[1])[:_N]
    pre = (agg + jnp.dot(x_ref[...], w_ref[...],
                         preferred_element_type=jnp.float32) + b_ref[...])
    m = jnp.mean(pre, axis=0, keepdims=True)
    v = jnp.mean((pre - m) * (pre - m), axis=0, keepdims=True)
    h = (pre - m) * lax.rsqrt(v + _EPS) * g_ref[...] + bt_ref[...]
    out_ref[...] = jnp.maximum(h, 0.0)


def _bn2_body(aggp_ref, aggq_ref, h_ref, w_ref, b_ref, g_ref, bt_ref,
              fcw_ref, fcb_ref, out_ref):
    agg = (aggp_ref[0] + aggp_ref[1] + aggq_ref[0] + aggq_ref[1])[:_N]
    pre = (agg + jnp.dot(h_ref[...], w_ref[...],
                         preferred_element_type=jnp.float32) + b_ref[...])
    m = jnp.mean(pre, axis=0, keepdims=True)
    v = jnp.mean((pre - m) * (pre - m), axis=0, keepdims=True)
    h2 = (pre - m) * lax.rsqrt(v + _EPS) * g_ref[...] + bt_ref[...]
    h2 = jnp.maximum(h2, 0.0)
    logit = jnp.dot(h2, fcw_ref[...],
                    preferred_element_type=jnp.float32) + fcb_ref[...]
    out_ref[...] = 1.0 / (1.0 + jnp.exp(-logit))


def _bn1(aggp, aggq, x, w, b, g, bt):
    return pl.pallas_call(
        _bn1_body,
        out_shape=jax.ShapeDtypeStruct((_N, _H), jnp.float32),
    )(aggp, aggq, x, w, b.reshape(1, _H), g.reshape(1, _H),
      bt.reshape(1, _H))


def _bn2fc(aggp, aggq, h, w, b, g, bt, fcw, fcb):
    return pl.pallas_call(
        _bn2_body,
        out_shape=jax.ShapeDtypeStruct((_N, 1), jnp.float32),
    )(aggp, aggq, h, w, b.reshape(1, _H), g.reshape(1, _H),
      bt.reshape(1, _H), fcw, fcb.reshape(1, 1))


def kernel(x, edge_index, edge_attr, en1_W, en1_b, root1_W, bias1, en2_W,
           en2_b, root2_W, bias2, bn1_gamma, bn1_beta, bn2_gamma, bn2_beta,
           fc_W, fc_b):
    eh = _E // 2
    pad_h = _EH - eh
    srcs, dsts, eas = [], [], []
    for lo in (0, eh):
        srcs.append(jnp.pad(lax.dynamic_slice_in_dim(edge_index[0], lo, eh),
                            (0, pad_h)).reshape(_NW, _CPW2, _CHUNK))
        dsts.append(jnp.pad(lax.dynamic_slice_in_dim(edge_index[1], lo, eh),
                            (0, pad_h),
                            constant_values=_N).reshape(_NW, _CPW2, _CHUNK))
        eas.append(jnp.pad(lax.dynamic_slice_in_dim(edge_attr, lo, eh),
                           ((0, pad_h), (0, 0))).reshape(
                               _EQ2, 4 * _F_E).astype(jnp.bfloat16))
    xpad = jnp.pad(x, ((0, 0), (0, _D - _F_IN)))
    zeros = jnp.zeros((_RPS, _D), jnp.float32)
    eye4 = jnp.eye(4, dtype=jnp.float32)
    w4_1 = jnp.kron(eye4, en1_W).astype(jnp.bfloat16)
    b4_1 = jnp.tile(en1_b, 4).reshape(1, -1)
    w4_2 = jnp.kron(eye4, en2_W).astype(jnp.bfloat16)
    b4_2 = jnp.tile(en2_b, 4).reshape(1, -1)

    def layer(table, w4, b4, r, s):
        xga = _gather(table, srcs[0])
        xgb = _gather(table, srcs[1])
        msga = _fused_msgs(eas[0], xga.reshape(_EQ2, 128), w4, b4, r, s)
        msgb = _fused_msgs(eas[1], xgb.reshape(_EQ2, 128), w4, b4, r, s)
        agga = _scatter(msga.reshape(_EH, _D), dsts[0], zeros)
        aggb = _scatter(msgb.reshape(_EH, _D), dsts[1], zeros)
        return agga, aggb

    agg1a, agg1b = layer(xpad, w4_1, b4_1, _R1, _S1)
    h = _bn1(agg1a, agg1b, x, root1_W, bias1, bn1_gamma, bn1_beta)
    agg2a, agg2b = layer(h, w4_2, b4_2, _R2, _S2)
    return _bn2fc(agg2a, agg2b, h, root2_W, bias2, bn2_gamma, bn2_beta,
                  fc_W, fc_b)


# speedup vs baseline: 1.3687x; 1.0272x over previous
import functools

import jax
import jax.numpy as jnp
import numpy as np
from jax import lax
from jax.experimental import pallas as pl
from jax.experimental.pallas import tpu as pltpu
from jax.experimental.pallas import tpu_sc as plsc

_N = 10000
_E = 160000
_F_IN = 22
_F_E = 4
_H = 32
_EPS = 1e-5
_D = 32
_NPAD = 10240

_NC = 2
_NS = 16
_NW = _NC * _NS
_CHUNK = 128
_CPW = 40
_EPW = _CHUNK * _CPW
_EP = _EPW * _NW
_EQ = _EP // 4
_QPC = _CHUNK // 4
_RPS = _NPAD // _NS

_EH = _EP // 2
_CPW2 = _CPW // 2
_EQ2 = _EH // 4

_BLKQ = 1024
_NBLK = _EQ2 // _BLKQ


def _expand_mats(in_ch):
    ch = in_ch * _H
    r = np.zeros((128, 4 * ch), np.float32)
    s = np.zeros((4 * ch, 128), np.float32)
    for q in range(4):
        for i in range(in_ch):
            r[q * _D + i, q * ch + i * _H:q * ch + (i + 1) * _H] = 1.0
            s[q * ch + i * _H:q * ch + (i + 1) * _H,
              q * _D:q * _D + _H] = np.eye(_H, dtype=np.float32)
    return r.astype(jnp.bfloat16), s.astype(jnp.bfloat16)


_R1, _S1 = _expand_mats(_F_IN)
_R2, _S2 = _expand_mats(_H)


@functools.cache
def _sc_mesh():
    return plsc.VectorSubcoreMesh(core_axis_name="c", subcore_axis_name="s",
                                  num_cores=_NC, num_subcores=_NS)


@functools.cache
def _gather_kernel(cpw):
    def body(table_hbm, idx_hbm, out_hbm, idx_v, rows0, rows1, sem0, sem1):
        wid = lax.axis_index("s") * _NC + lax.axis_index("c")
        pltpu.sync_copy(idx_hbm.at[wid], idx_v)
        base = wid * cpw * _CHUNK
        bufs = ((rows0, sem0), (rows1, sem1))
        pltpu.async_copy(table_hbm.at[idx_v.at[0]], rows0, sem0)
        pltpu.async_copy(table_hbm.at[idx_v.at[1]], rows1, sem1)

        def pair(k, carry):
            for b in range(2):
                rows, sem = bufs[b]
                g = 2 * k + b
                pltpu.make_async_copy(table_hbm.at[idx_v.at[g]], rows,
                                      sem).wait()
                pltpu.sync_copy(rows,
                                out_hbm.at[pl.ds(base + g * _CHUNK, _CHUNK)])

                @pl.when(g + 2 < cpw)
                def _(rows=rows, sem=sem, g=g):
                    pltpu.async_copy(table_hbm.at[idx_v.at[g + 2]], rows, sem)

            return carry

        lax.fori_loop(0, cpw // 2, pair, 0)

    return pl.kernel(
        body,
        out_type=jax.ShapeDtypeStruct((_NW * cpw * _CHUNK, _D), jnp.float32),
        mesh=_sc_mesh(),
        compiler_params=pltpu.CompilerParams(use_tc_tiling_on_sc=False),
        scratch_types=[
            pltpu.VMEM((cpw, _CHUNK), jnp.int32),
            pltpu.VMEM((_CHUNK, _D), jnp.float32),
            pltpu.VMEM((_CHUNK, _D), jnp.float32),
            pltpu.SemaphoreType.DMA,
            pltpu.SemaphoreType.DMA,
        ],
    )


def _gather(table, idx):
    return _gather_kernel(idx.shape[1])(table, idx)


@functools.cache
def _scatter_kernel(cpw):
    def body(msg_hbm, idx_hbm, zeros_hbm, out_hbm, idx_v, rows0, rows1,
             acc_sh, sem0, sem1):
        cid = lax.axis_index("c")
        sid = lax.axis_index("s")
        wid = sid * _NC + cid
        pltpu.sync_copy(zeros_hbm, acc_sh.at[pl.ds(sid * _RPS, _RPS)])
        plsc.subcore_barrier()
        pltpu.sync_copy(idx_hbm.at[wid], idx_v)
        base = wid * cpw * _CHUNK
        bufs = ((rows0, sem0), (rows1, sem1))
        pltpu.async_copy(msg_hbm.at[pl.ds(base, _CHUNK)], rows0, sem0)
        pltpu.async_copy(msg_hbm.at[pl.ds(base + _CHUNK, _CHUNK)], rows1,
                         sem1)

        def pair(k, carry):
            for b in range(2):
                rows, sem = bufs[b]
                g = 2 * k + b
                pltpu.make_async_copy(
                    msg_hbm.at[pl.ds(base + g * _CHUNK, _CHUNK)], rows,
                    sem).wait()
                pltpu.sync_copy(rows, acc_sh.at[idx_v.at[g]], add=True)

                @pl.when(g + 2 < cpw)
                def _(rows=rows, sem=sem, g=g):
                    pltpu.async_copy(
                        msg_hbm.at[pl.ds(base + (g + 2) * _CHUNK, _CHUNK)],
                        rows, sem)

            return carry

        lax.fori_loop(0, cpw // 2, pair, 0)
        plsc.subcore_barrier()
        pltpu.sync_copy(acc_sh.at[pl.ds(sid * _RPS, _RPS)],
                        out_hbm.at[cid, pl.ds(sid * _RPS, _RPS)])

    return pl.kernel(
        body,
        out_type=jax.ShapeDtypeStruct((_NC, _NPAD, _D), jnp.float32),
        mesh=_sc_mesh(),
        compiler_params=pltpu.CompilerParams(use_tc_tiling_on_sc=False),
        scratch_types=[
            pltpu.VMEM((cpw, _CHUNK), jnp.int32),
            pltpu.VMEM((_CHUNK, _D), jnp.float32),
            pltpu.VMEM((_CHUNK, _D), jnp.float32),
            pltpu.VMEM_SHARED((_NPAD, _D), jnp.float32),
            pltpu.SemaphoreType.DMA,
            pltpu.SemaphoreType.DMA,
        ],
    )


def _scatter(msg, idx, zeros):
    return _scatter_kernel(idx.shape[1])(msg, idx, zeros)


def _fused_body(ea4_ref, xg4_ref, w4_ref, r4_ref, s4_ref, out_ref):
    z = jnp.maximum(
        jnp.dot(ea4_ref[...], w4_ref[...],
                preferred_element_type=jnp.float32), 0.0)
    xr = jnp.dot(xg4_ref[...].astype(jnp.bfloat16), r4_ref[...],
                 preferred_element_type=jnp.float32)
    p = (z * xr).astype(jnp.bfloat16)
    out_ref[...] = jnp.dot(p, s4_ref[...],
                           preferred_element_type=jnp.float32)


def _fused_msgs(ea4, xg4, w4, r4, s4):
    ke, ch4 = w4.shape
    return pl.pallas_call(
        _fused_body,
        grid=(_NBLK,),
        in_specs=[
            pl.BlockSpec((_BLKQ, ke), lambda i: (i, 0)),
            pl.BlockSpec((_BLKQ, 128), lambda i: (i, 0)),
            pl.BlockSpec((ke, ch4), lambda i: (0, 0)),
            pl.BlockSpec((128, ch4), lambda i: (0, 0)),
            pl.BlockSpec((ch4, 128), lambda i: (0, 0)),
        ],
        out_specs=pl.BlockSpec((_BLKQ, 128), lambda i: (i, 0)),
        out_shape=jax.ShapeDtypeStruct((_EQ2, 128), jnp.float32),
    )(ea4, xg4, w4, r4, s4)


def _bn1_body(aggp_ref, aggq_ref, x_ref, w_ref, b_ref, g_ref, bt_ref,
              out_ref):
    agg = (aggp_ref[0] + aggp_ref[1] + aggq_ref[0] + aggq_ref[1])[:_N]
    pre = (agg + jnp.dot(x_ref[...], w_ref[...],
                         preferred_element_type=jnp.float32) + b_ref[...])
    m = jnp.mean(pre, axis=0, keepdims=True)
    v = jnp.mean((pre - m) * (pre - m), axis=0, keepdims=True)
    h = (pre - m) * lax.rsqrt(v + _EPS) * g_ref[...] + bt_ref[...]
    out_ref[...] = jnp.maximum(h, 0.0)


def _bn2_body(aggp_ref, aggq_ref, h_ref, w_ref, b_ref, g_ref, bt_ref,
              fcw_ref, fcb_ref, out_ref):
    agg = (aggp_ref[0] + aggp_ref[1] + aggq_ref[0] + aggq_ref[1])[:_N]
    pre = (agg + jnp.dot(h_ref[...], w_ref[...],
                         preferred_element_type=jnp.float32) + b_ref[...])
    m = jnp.mean(pre, axis=0, keepdims=True)
    v = jnp.mean((pre - m) * (pre - m), axis=0, keepdims=True)
    h2 = (pre - m) * lax.rsqrt(v + _EPS) * g_ref[...] + bt_ref[...]
    h2 = jnp.maximum(h2, 0.0)
    logit = jnp.dot(h2, fcw_ref[...],
                    preferred_element_type=jnp.float32) + fcb_ref[...]
    out_ref[...] = 1.0 / (1.0 + jnp.exp(-logit))


def _bn1(aggp, aggq, x, w, b, g, bt):
    return pl.pallas_call(
        _bn1_body,
        out_shape=jax.ShapeDtypeStruct((_N, _H), jnp.float32),
    )(aggp, aggq, x, w, b.reshape(1, _H), g.reshape(1, _H),
      bt.reshape(1, _H))


def _bn2fc(aggp, aggq, h, w, b, g, bt, fcw, fcb):
    return pl.pallas_call(
        _bn2_body,
        out_shape=jax.ShapeDtypeStruct((_N, 1), jnp.float32),
    )(aggp, aggq, h, w, b.reshape(1, _H), g.reshape(1, _H),
      bt.reshape(1, _H), fcw, fcb.reshape(1, 1))


def kernel(x, edge_index, edge_attr, en1_W, en1_b, root1_W, bias1, en2_W,
           en2_b, root2_W, bias2, bn1_gamma, bn1_beta, bn2_gamma, bn2_beta,
           fc_W, fc_b):
    eh = _E // 2
    pad_h = _EH - eh
    srcs, dsts, eas = [], [], []
    for lo in (0, eh):
        srcs.append(jnp.pad(lax.dynamic_slice_in_dim(edge_index[0], lo, eh),
                            (0, pad_h)).reshape(_NW, _CPW2, _CHUNK))
        dsts.append(jnp.pad(lax.dynamic_slice_in_dim(edge_index[1], lo, eh),
                            (0, pad_h),
                            constant_values=_N).reshape(_NW, _CPW2, _CHUNK))
        ea_half = jnp.pad(lax.dynamic_slice_in_dim(edge_attr, lo, eh),
                          ((0, pad_h), (0, 0))).reshape(
                              _EQ2, 4 * _F_E).astype(jnp.bfloat16)
        eas.append(jnp.concatenate(
            [ea_half, jnp.ones((_EQ2, 4), jnp.bfloat16)], axis=1))
    xpad = jnp.pad(x, ((0, 0), (0, _D - _F_IN)))
    zeros = jnp.zeros((_RPS, _D), jnp.float32)
    eye4 = jnp.eye(4, dtype=jnp.float32)

    def _w4(nn_w, nn_b):
        w4 = jnp.kron(eye4, nn_w)
        b4 = jnp.kron(eye4, nn_b.reshape(1, -1))
        return jnp.concatenate([w4, b4], axis=0).astype(jnp.bfloat16)

    w4_1 = _w4(en1_W, en1_b)
    w4_2 = _w4(en2_W, en2_b)

    def layer(table, w4, r, s):
        xga = _gather(table, srcs[0])
        xgb = _gather(table, srcs[1])
        msga = _fused_msgs(eas[0], xga.reshape(_EQ2, 128), w4, r, s)
        msgb = _fused_msgs(eas[1], xgb.reshape(_EQ2, 128), w4, r, s)
        agga = _scatter(msga.reshape(_EH, _D), dsts[0], zeros)
        aggb = _scatter(msgb.reshape(_EH, _D), dsts[1], zeros)
        return agga, aggb

    agg1a, agg1b = layer(xpad, w4_1, _R1, _S1)
    h = _bn1(agg1a, agg1b, x, root1_W, bias1, bn1_gamma, bn1_beta)
    agg2a, agg2b = layer(h, w4_2, _R2, _S2)
    return _bn2fc(agg2a, agg2b, h, root2_W, bias2, bn2_gamma, bn2_beta,
                  fc_W, fc_b)
